# Initial kernel scaffold; baseline (speedup 1.0000x reference)
import functools
import math

import jax
import jax.numpy as jnp
from jax import lax
from jax.experimental import pallas as pl
from jax.experimental.pallas import tpu as pltpu
from jax.experimental.pallas import tpu_sc as plsc

_BN_EPS = 1e-5
_NBT = 6

_NC = 2
_NS = 16
_NW = _NC * _NS
_CHUNK = 128


def _sc_message_agg(N, E, D):
    n_chunks = E // _CHUNK
    assert n_chunks * _CHUNK == E
    iters = (n_chunks + _NW - 1) // _NW
    rows_per_tile = N // _NS
    assert rows_per_tile * _NS == N
    full, tail = divmod(rows_per_tile, _CHUNK)

    mesh = plsc.VectorSubcoreMesh(core_axis_name="c", subcore_axis_name="s")

    @functools.partial(
        pl.kernel,
        out_type=jax.ShapeDtypeStruct((_NC, N, D), jnp.float32),
        mesh=mesh,
        scratch_types=[
            pltpu.VMEM((1, _CHUNK), jnp.int32),
            pltpu.VMEM((1, _CHUNK), jnp.int32),
            pltpu.VMEM((1, _CHUNK), jnp.int32),
            pltpu.VMEM((_CHUNK, D), jnp.float32),
            pltpu.VMEM((_CHUNK, D), jnp.float32),
            pltpu.VMEM_SHARED((N, D), jnp.float32),
            pltpu.SemaphoreType.DMA,
            pltpu.SemaphoreType.DMA,
        ],
    )
    def sc_kernel(h_hbm, ei_hbm, cidx_hbm, ctab_hbm, out_hbm,
                  srcv, dstv, cv, buf_a, buf_b, agg, sem_a, sem_b):
        c = lax.axis_index("c")
        s = lax.axis_index("s")
        wid = s * _NC + c

        def zrow(i, carry):
            for j in range(D // 16):
                buf_a[i, pl.ds(j * 16, 16)] = jnp.zeros((16,), jnp.float32)
            return carry

        lax.fori_loop(0, _CHUNK, zrow, 0)
        base = s * rows_per_tile
        for t in range(full):
            pltpu.sync_copy(buf_a, agg.at[pl.ds(base + t * _CHUNK, _CHUNK)])
        if tail:
            pltpu.sync_copy(buf_a.at[pl.ds(0, tail)],
                            agg.at[pl.ds(base + full * _CHUNK, tail)])
        plsc.subcore_barrier()

        def chunk(it, carry):
            ci = it * _NW + wid

            @pl.when(ci < n_chunks)
            def _():
                e0 = ci * _CHUNK
                pltpu.sync_copy(ei_hbm.at[0, pl.ds(e0, _CHUNK)], srcv.at[0])
                pltpu.sync_copy(ei_hbm.at[1, pl.ds(e0, _CHUNK)], dstv.at[0])
                pltpu.sync_copy(cidx_hbm.at[pl.ds(e0, _CHUNK)], cv.at[0])
                ga = pltpu.async_copy(ctab_hbm.at[cv.at[0]], buf_a, sem_a)
                gb = pltpu.async_copy(h_hbm.at[srcv.at[0]], buf_b, sem_b)
                ga.wait()
                gb.wait()

                def row(i, cc):
                    for j in range(D // 16):
                        sl = pl.ds(j * 16, 16)
                        buf_a[i, sl] = jnp.maximum(buf_a[i, sl] + buf_b[i, sl], 0.0)
                    return cc

                lax.fori_loop(0, _CHUNK, row, 0)
                pltpu.sync_copy(buf_a, agg.at[dstv.at[0]], add=True)

            return carry

        lax.fori_loop(0, iters, chunk, 0)
        plsc.subcore_barrier()

        for t in range(full):
            sl = pl.ds(base + t * _CHUNK, _CHUNK)
            pltpu.sync_copy(agg.at[sl], out_hbm.at[c, sl])
        if tail:
            sl = pl.ds(base + full * _CHUNK, tail)
            pltpu.sync_copy(agg.at[sl], out_hbm.at[c, sl])

    return sc_kernel


def _tc_mlp(N, D, relu_out):
    blk = 2000
    assert N % blk == 0

    def body(h, a0, a1, ep, w1, b1, w2, b2, o):
        z = h[...] * ep[...] + a0[...] + a1[...]
        u = jnp.dot(z, w1[...], preferred_element_type=jnp.float32) + b1[...]
        u = jnp.maximum(u, 0.0)
        y = jnp.dot(u, w2[...], preferred_element_type=jnp.float32) + b2[...]
        if relu_out:
            y = jnp.maximum(y, 0.0)
        o[...] = y

    row_spec = pl.BlockSpec((blk, D), lambda i: (i, 0))
    return pl.pallas_call(
        body,
        grid=(N // blk,),
        in_specs=[
            row_spec,
            row_spec,
            row_spec,
            pl.BlockSpec((1, D), lambda i: (0, 0)),
            pl.BlockSpec((D, 2 * D), lambda i: (0, 0)),
            pl.BlockSpec((1, 2 * D), lambda i: (0, 0)),
            pl.BlockSpec((2 * D, D), lambda i: (0, 0)),
            pl.BlockSpec((1, D), lambda i: (0, 0)),
        ],
        out_specs=row_spec,
        out_shape=jax.ShapeDtypeStruct((N, D), jnp.float32),
    )


def kernel(x, edge_index, edge_attr, bond_w, W1, b1, g1, be1, W2, b2, eps, bng, bnb):
    N, D = x.shape
    E = edge_index.shape[1]
    L = bond_w.shape[0]
    kbn = 1.0 / math.sqrt(1.0 + _BN_EPS)

    cidx = (edge_attr[:, 0] * 3 + edge_attr[:, 1]).astype(jnp.int32)
    sc = _sc_message_agg(N, E, D)

    h = x
    for l in range(L):
        emb = bond_w[l].T
        ctab = (emb[:_NBT, None, :] + emb[None, _NBT:, :]).reshape(-1, D)
        agg2 = sc(h, edge_index, cidx, ctab)
        s1 = kbn * g1[l]
        w1t = W1[l].T * s1[None, :]
        bias1 = (b1[l] * s1 + be1[l])[None, :]
        s2 = kbn * bng[l]
        w2t = W2[l].T * s2[None, :]
        bias2 = (b2[l] * s2 + bnb[l])[None, :]
        ep = jnp.full((1, D), 1.0 + eps[l], jnp.float32)
        h = _tc_mlp(N, D, l < L - 1)(h, agg2[0], agg2[1], ep, w1t, bias1, w2t, bias2)
    return h


# speedup vs baseline: 1.7255x; 1.7255x over previous
import functools
import math

import jax
import jax.numpy as jnp
from jax import lax
from jax.experimental import pallas as pl
from jax.experimental.pallas import tpu as pltpu
from jax.experimental.pallas import tpu_sc as plsc

_BN_EPS = 1e-5
_NBT = 6

_NC = 2
_NS = 16
_NW = _NC * _NS
_CHUNK = 128


def _sc_message_agg(N, E, D):
    n_chunks = E // _CHUNK
    assert n_chunks * _CHUNK == E
    iters = (n_chunks + _NW - 1) // _NW
    rpt = (N // (8 * _NS)) * 8
    rem = N - rpt * _NS
    full, tail = divmod(rpt, _CHUNK)

    mesh = plsc.VectorSubcoreMesh(core_axis_name="c", subcore_axis_name="s")

    @functools.partial(
        pl.kernel,
        out_type=jax.ShapeDtypeStruct((_NC, N, D), jnp.float32),
        mesh=mesh,
        scratch_types=[
            pltpu.VMEM((1, _CHUNK), jnp.int32),
            pltpu.VMEM((1, _CHUNK), jnp.int32),
            pltpu.VMEM((1, _CHUNK), jnp.int32),
            pltpu.VMEM((_CHUNK, D), jnp.float32),
            pltpu.VMEM((_CHUNK, D), jnp.float32),
            pltpu.VMEM_SHARED((N, D), jnp.float32),
            pltpu.SemaphoreType.DMA,
            pltpu.SemaphoreType.DMA,
        ],
    )
    def sc_kernel(h_hbm, ei_hbm, cidx_hbm, ctab_hbm, out_hbm,
                  srcv, dstv, cv, buf_a, buf_b, agg, sem_a, sem_b):
        c = lax.axis_index("c")
        s = lax.axis_index("s")
        wid = s * _NC + c

        def zrow(i, carry):
            for j in range(D // 16):
                buf_a[i, pl.ds(j * 16, 16)] = jnp.zeros((16,), jnp.float32)
            return carry

        lax.fori_loop(0, _CHUNK, zrow, 0)
        base = s * rpt
        for t in range(full):
            pltpu.sync_copy(buf_a, agg.at[pl.ds(base + t * _CHUNK, _CHUNK)])
        if tail:
            pltpu.sync_copy(buf_a.at[pl.ds(0, tail)],
                            agg.at[pl.ds(base + full * _CHUNK, tail)])
        if rem:

            @pl.when(s == _NS - 1)
            def _():
                pltpu.sync_copy(buf_a.at[pl.ds(0, rem)],
                                agg.at[pl.ds(rpt * _NS, rem)])

        plsc.subcore_barrier()

        def chunk(it, carry):
            ci = it * _NW + wid

            @pl.when(ci < n_chunks)
            def _():
                e0 = ci * _CHUNK
                pltpu.sync_copy(ei_hbm.at[0, pl.ds(e0, _CHUNK)], srcv.at[0])
                pltpu.sync_copy(ei_hbm.at[1, pl.ds(e0, _CHUNK)], dstv.at[0])
                pltpu.sync_copy(cidx_hbm.at[pl.ds(e0, _CHUNK)], cv.at[0])
                ga = pltpu.async_copy(ctab_hbm.at[cv.at[0]], buf_a, sem_a)
                gb = pltpu.async_copy(h_hbm.at[srcv.at[0]], buf_b, sem_b)
                ga.wait()
                gb.wait()

                def row(i, cc):
                    for j in range(D // 16):
                        sl = pl.ds(j * 16, 16)
                        buf_a[i, sl] = jnp.maximum(buf_a[i, sl] + buf_b[i, sl], 0.0)
                    return cc

                lax.fori_loop(0, _CHUNK, row, 0)
                pltpu.sync_copy(buf_a, agg.at[dstv.at[0]], add=True)

            return carry

        lax.fori_loop(0, iters, chunk, 0)
        plsc.subcore_barrier()

        for t in range(full):
            sl = pl.ds(base + t * _CHUNK, _CHUNK)
            pltpu.sync_copy(agg.at[sl], out_hbm.at[c, sl])
        if tail:
            sl = pl.ds(base + full * _CHUNK, tail)
            pltpu.sync_copy(agg.at[sl], out_hbm.at[c, sl])
        if rem:

            @pl.when(s == _NS - 1)
            def _():
                sl = pl.ds(rpt * _NS, rem)
                pltpu.sync_copy(agg.at[sl], out_hbm.at[c, sl])

    return sc_kernel


def _tc_mlp(N, D, relu_out):
    blk = 2000
    assert N % blk == 0

    def body(h, a0, a1, ep, w1, b1, w2, b2, o):
        z = h[...] * ep[...] + a0[...] + a1[...]
        u = jnp.dot(z, w1[...], preferred_element_type=jnp.float32) + b1[...]
        u = jnp.maximum(u, 0.0)
        y = jnp.dot(u, w2[...], preferred_element_type=jnp.float32) + b2[...]
        if relu_out:
            y = jnp.maximum(y, 0.0)
        o[...] = y

    row_spec = pl.BlockSpec((blk, D), lambda i: (i, 0))
    return pl.pallas_call(
        body,
        grid=(N // blk,),
        in_specs=[
            row_spec,
            row_spec,
            row_spec,
            pl.BlockSpec((1, D), lambda i: (0, 0)),
            pl.BlockSpec((D, 2 * D), lambda i: (0, 0)),
            pl.BlockSpec((1, 2 * D), lambda i: (0, 0)),
            pl.BlockSpec((2 * D, D), lambda i: (0, 0)),
            pl.BlockSpec((1, D), lambda i: (0, 0)),
        ],
        out_specs=row_spec,
        out_shape=jax.ShapeDtypeStruct((N, D), jnp.float32),
    )


def kernel(x, edge_index, edge_attr, bond_w, W1, b1, g1, be1, W2, b2, eps, bng, bnb):
    N, D = x.shape
    E = edge_index.shape[1]
    L = bond_w.shape[0]
    kbn = 1.0 / math.sqrt(1.0 + _BN_EPS)

    cidx = (edge_attr[:, 0] * 3 + edge_attr[:, 1]).astype(jnp.int32)
    sc = _sc_message_agg(N, E, D)

    h = x
    for l in range(L):
        emb = bond_w[l].T
        ctab = (emb[:_NBT, None, :] + emb[None, _NBT:, :]).reshape(-1, D)
        agg2 = sc(h, edge_index, cidx, ctab)
        s1 = kbn * g1[l]
        w1t = W1[l].T * s1[None, :]
        bias1 = (b1[l] * s1 + be1[l])[None, :]
        s2 = kbn * bng[l]
        w2t = W2[l].T * s2[None, :]
        bias2 = (b2[l] * s2 + bnb[l])[None, :]
        ep = jnp.full((1, D), 1.0 + eps[l], jnp.float32)
        h = _tc_mlp(N, D, l < L - 1)(h, agg2[0], agg2[1], ep, w1t, bias1, w2t, bias2)
    return h
